# Initial kernel scaffold; baseline (speedup 1.0000x reference)
import jax
import jax.numpy as jnp
from jax.experimental import pallas as pl


def kernel(x, edge_index, batch, other_attrs, Wl0, bl0, Wr0, g0, be0, Wl1, bl1, Wr1, g1, be1, Wl2, bl2, Wr2, g2, be2, W1, b1, W2, b2, W3, b3, Rw1, Rb1, Rw2, Rb2):
    raise NotImplementedError("write your pallas kernel here")


# speedup vs baseline: 3.6916x; 3.6916x over previous
import functools
import math

import jax
import jax.numpy as jnp
from jax import lax
from jax.experimental import pallas as pl
from jax.experimental.pallas import tpu as pltpu
from jax.experimental.pallas import tpu_sc as plsc

N = 10000
F = 128
H = 128
G = 64
NSP = 10240
CHUNK = 128
NW = 32
ROWS_PER_SUB = NSP // 16
BN_SCALE = 1.0 / math.sqrt(1.0 + 1e-5)
R = 2000


def _make_sc_agg(per_worker: int, with_count: bool):
  mesh = plsc.VectorSubcoreMesh(core_axis_name="c", subcore_axis_name="s")
  out_type = [jax.ShapeDtypeStruct((2 * NSP, H), jnp.float32)]
  scratch = [
      pltpu.VMEM((CHUNK,), jnp.int32),
      pltpu.VMEM((CHUNK,), jnp.int32),
      pltpu.VMEM((CHUNK, H), jnp.float32),
      pltpu.VMEM_SHARED((NSP, H), jnp.float32),
      pltpu.SemaphoreType.DMA,
  ]
  if with_count:
    out_type.append(jax.ShapeDtypeStruct((2 * NSP,), jnp.float32))
    scratch += [
        pltpu.VMEM((CHUNK,), jnp.float32),
        pltpu.VMEM_SHARED((NSP,), jnp.float32),
    ]

  def body(h_hbm, src_hbm, dst_hbm, zr2_hbm, zr1_hbm, *rest):
    if with_count:
      agg_out, cnt_out, sidx, didx, rows, agg_sh, sem, ones, cnt_sh = rest
    else:
      agg_out, sidx, didx, rows, agg_sh, sem = rest
    c = lax.axis_index("c")
    s = lax.axis_index("s")
    wid = s * 2 + c
    r0 = s * ROWS_PER_SUB
    pltpu.sync_copy(zr2_hbm.at[pl.ds(r0, ROWS_PER_SUB)],
                    agg_sh.at[pl.ds(r0, ROWS_PER_SUB)])
    if with_count:
      pltpu.sync_copy(zr1_hbm.at[pl.ds(r0, ROWS_PER_SUB)],
                      cnt_sh.at[pl.ds(r0, ROWS_PER_SUB)])
      for j in range(CHUNK // 16):
        ones[pl.ds(j * 16, 16)] = jnp.ones((16,), jnp.float32)
    plsc.subcore_barrier()

    def step(i, carry):
      base = (wid * per_worker + i) * CHUNK
      pltpu.sync_copy(src_hbm.at[pl.ds(base, CHUNK)], sidx)
      pltpu.sync_copy(dst_hbm.at[pl.ds(base, CHUNK)], didx)
      pltpu.async_copy(h_hbm.at[sidx], rows, sem).wait()
      pltpu.sync_copy(rows, agg_sh.at[didx], add=True)
      if with_count:
        pltpu.sync_copy(ones, cnt_sh.at[didx], add=True)
      return carry

    lax.fori_loop(0, per_worker, step, 0)
    plsc.subcore_barrier()
    out_base = c * NSP + r0
    pltpu.sync_copy(agg_sh.at[pl.ds(r0, ROWS_PER_SUB)],
                    agg_out.at[pl.ds(out_base, ROWS_PER_SUB)])
    if with_count:
      pltpu.sync_copy(cnt_sh.at[pl.ds(r0, ROWS_PER_SUB)],
                      cnt_out.at[pl.ds(out_base, ROWS_PER_SUB)])

  return pl.kernel(body, out_type=out_type, mesh=mesh, scratch_types=scratch)


def _tc_layer_body(agg0, agg1, cnt0, cnt1, h, wl, wr, bl, g, be, out):
  cnt = cnt0[...] + cnt1[...]
  denom = jnp.maximum(cnt, 1.0)
  agg = (agg0[...] + agg1[...]) / denom
  o = jnp.dot(agg, wl[...], preferred_element_type=jnp.float32)
  o = o + jnp.dot(h[...], wr[...], preferred_element_type=jnp.float32)
  o = o + bl[...]
  o = g[...] * (o * BN_SCALE) + be[...]
  out[...] = jnp.maximum(o, 0.0)


_tc_layer = pl.pallas_call(
    _tc_layer_body,
    grid=(N // R,),
    in_specs=[
        pl.BlockSpec((R, H), lambda i: (i, 0)),
        pl.BlockSpec((R, H), lambda i: (i, 0)),
        pl.BlockSpec((R, 1), lambda i: (i, 0)),
        pl.BlockSpec((R, 1), lambda i: (i, 0)),
        pl.BlockSpec((R, H), lambda i: (i, 0)),
        pl.BlockSpec((H, H), lambda i: (0, 0)),
        pl.BlockSpec((H, H), lambda i: (0, 0)),
        pl.BlockSpec((1, H), lambda i: (0, 0)),
        pl.BlockSpec((1, H), lambda i: (0, 0)),
        pl.BlockSpec((1, H), lambda i: (0, 0)),
    ],
    out_specs=pl.BlockSpec((R, H), lambda i: (i, 0)),
    out_shape=jax.ShapeDtypeStruct((N, H), jnp.float32),
)


def _head_body(h0, h1, h2, x1, batchf, oa17,
               w1a, w1b, w1c, b1, w2, b2, w3, b3,
               rw1a, rw1b, rw1c0, rw1c1, rw1c2, rb1, rw2, rb2,
               node_out, reg_out,
               pool0, pool1, pool2, cls_acc, cnt_acc):
  i = pl.program_id(0)
  nb = pl.num_programs(0)
  a0, a1, a2 = h0[...], h1[...], h2[...]
  t = jnp.dot(a0, w1a[...], preferred_element_type=jnp.float32)
  t = t + jnp.dot(a1, w1b[...], preferred_element_type=jnp.float32)
  t = t + jnp.dot(a2, w1c[...], preferred_element_type=jnp.float32)
  t = jnp.maximum(t + b1[...], 0.0)
  t = jnp.maximum(jnp.dot(t, w2[...], preferred_element_type=jnp.float32)
                  + b2[...], 0.0)
  no = jnp.dot(t, w3[...], preferred_element_type=jnp.float32) + b3[...]
  node_out[...] = no
  nd = no * x1[...]

  bb = batchf[...]
  iota = lax.broadcasted_iota(jnp.int32, (R, G), 1)
  P = (bb == iota).astype(jnp.float32)

  @pl.when(i == 0)
  def _():
    pool0[...] = jnp.zeros_like(pool0)
    pool1[...] = jnp.zeros_like(pool1)
    pool2[...] = jnp.zeros_like(pool2)
    cls_acc[...] = jnp.zeros_like(cls_acc)
    cnt_acc[...] = jnp.zeros_like(cnt_acc)

  dn = (((0,), (0,)), ((), ()))
  pool0[...] += lax.dot_general(P, a0, dn, preferred_element_type=jnp.float32)
  pool1[...] += lax.dot_general(P, a1, dn, preferred_element_type=jnp.float32)
  pool2[...] += lax.dot_general(P, a2, dn, preferred_element_type=jnp.float32)
  cls_acc[...] += lax.dot_general(P, nd, dn, preferred_element_type=jnp.float32)
  cnt_acc[...] += jnp.sum(P, axis=0)[:, None]

  @pl.when(i == nb - 1)
  def _():
    gden = jnp.maximum(cnt_acc[...], 1.0)
    inv = 1.0 / gden
    x_class = cls_acc[...] * inv
    r = jnp.dot(oa17[...], rw1a[...], preferred_element_type=jnp.float32)
    r = r + jnp.dot(x_class, rw1b[...], preferred_element_type=jnp.float32)
    r = r + jnp.dot(pool0[...] * inv, rw1c0[...],
                    preferred_element_type=jnp.float32)
    r = r + jnp.dot(pool1[...] * inv, rw1c1[...],
                    preferred_element_type=jnp.float32)
    r = r + jnp.dot(pool2[...] * inv, rw1c2[...],
                    preferred_element_type=jnp.float32)
    r = jnp.maximum(r + rb1[...], 0.0)
    reg_out[...] = (jnp.dot(r, rw2[...], preferred_element_type=jnp.float32)
                    + rb2[...])


def _full(shape):
  return pl.BlockSpec(shape, lambda i: tuple(0 for _ in shape))


_head = pl.pallas_call(
    _head_body,
    grid=(N // R,),
    in_specs=[
        pl.BlockSpec((R, H), lambda i: (i, 0)),
        pl.BlockSpec((R, H), lambda i: (i, 0)),
        pl.BlockSpec((R, H), lambda i: (i, 0)),
        pl.BlockSpec((R, 1), lambda i: (i, 0)),
        pl.BlockSpec((R, 1), lambda i: (i, 0)),
        _full((G, 17)),
        _full((H, 2 * H)), _full((H, 2 * H)), _full((H, 2 * H)),
        _full((1, 2 * H)),
        _full((2 * H, H // 2)), _full((1, H // 2)),
        _full((H // 2, 1)), _full((1, 1)),
        _full((17, 32)), _full((1, 32)),
        _full((H, 32)), _full((H, 32)), _full((H, 32)),
        _full((1, 32)),
        _full((32, 1)), _full((1, 1)),
    ],
    out_specs=[
        pl.BlockSpec((R, 1), lambda i: (i, 0)),
        pl.BlockSpec((G, 1), lambda i: (0, 0)),
    ],
    out_shape=[
        jax.ShapeDtypeStruct((N, 1), jnp.float32),
        jax.ShapeDtypeStruct((G, 1), jnp.float32),
    ],
    scratch_shapes=[
        pltpu.VMEM((G, H), jnp.float32),
        pltpu.VMEM((G, H), jnp.float32),
        pltpu.VMEM((G, H), jnp.float32),
        pltpu.VMEM((G, 1), jnp.float32),
        pltpu.VMEM((G, 1), jnp.float32),
    ],
)


def kernel(x, edge_index, batch, other_attrs,
           Wl0, bl0, Wr0, g0, be0,
           Wl1, bl1, Wr1, g1, be1,
           Wl2, bl2, Wr2, g2, be2,
           W1, b1, W2, b2, W3, b3, Rw1, Rb1, Rw2, Rb2):
  E = edge_index.shape[1]
  ops = -(-E // CHUNK)
  ops = -(-ops // NW) * NW
  e_pad = ops * CHUNK
  per_worker = ops // NW

  src = edge_index[0].astype(jnp.int32)
  dst = edge_index[1].astype(jnp.int32)
  pad = e_pad - E
  if pad:
    src = jnp.concatenate([src, jnp.zeros((pad,), jnp.int32)])
    dst = jnp.concatenate([dst, jnp.full((pad,), NSP - 1, jnp.int32)])
  zr2 = jnp.zeros((NSP, H), jnp.float32)
  zr1 = jnp.zeros((NSP,), jnp.float32)

  sc_agg_cnt = _make_sc_agg(per_worker, with_count=True)
  sc_agg = _make_sc_agg(per_worker, with_count=False)

  agg_p, cnt_p = sc_agg_cnt(x, src, dst, zr2, zr1)
  cnt0 = cnt_p[:N, None]
  cnt1 = cnt_p[NSP:NSP + N, None]

  def layer(h, aggp, Wl, bl, Wr, g, be):
    return _tc_layer(aggp[:N], aggp[NSP:NSP + N], cnt0, cnt1, h,
                     Wl, Wr, bl[None], g[None], be[None])

  h0 = layer(x, agg_p, Wl0, bl0, Wr0, g0, be0)
  agg_p1, = sc_agg(h0, src, dst, zr2, zr1)
  h1 = layer(h0, agg_p1, Wl1, bl1, Wr1, g1, be1)
  agg_p2, = sc_agg(h1, src, dst, zr2, zr1)
  h2 = layer(h1, agg_p2, Wl2, bl2, Wr2, g2, be2)

  node_output, reg_output = _head(
      h0, h1, h2, x[:, 0:1], batch.astype(jnp.int32)[:, None],
      other_attrs[:, :17],
      W1[:H], W1[H:2 * H], W1[2 * H:], b1[None],
      W2, b2[None], W3, b3[None],
      Rw1[:17], Rw1[17:18], Rw1[18:18 + H], Rw1[18 + H:18 + 2 * H],
      Rw1[18 + 2 * H:], Rb1[None], Rw2, Rb2[None])

  last_attr = other_attrs[:, -1:]
  return (node_output, reg_output, last_attr)
